# baseline (speedup 1.0000x reference)
import functools

import jax
import jax.numpy as jnp
from jax import lax
from jax.experimental import pallas as pl
from jax.experimental.pallas import tpu as pltpu
from jax.experimental.pallas import tpu_sc as plsc

N = 10000
E = 320000
DH = 256
DHH = 128
L = 4
NSUB = 16
KU = 100
UNITS = 200
K_DEG = 80
CPT_DEG = 125
STRIPE = 624
STRIPE_LAST = N - 15 * STRIPE

_mesh = plsc.VectorSubcoreMesh(core_axis_name="c", subcore_axis_name="s")
_f32 = jnp.float32


def _per_stripe(s, fn):
    @pl.when(s < 15)
    def _():
        fn(pl.ds(s * STRIPE, STRIPE))

    @pl.when(s == 15)
    def _():
        fn(pl.ds(15 * STRIPE, STRIPE_LAST))


@functools.partial(
    pl.kernel,
    out_type=(
        jax.ShapeDtypeStruct((N, DHH), _f32),
        jax.ShapeDtypeStruct((N, DHH), _f32),
    ),
    mesh=_mesh,
    scratch_types=[
        pltpu.VMEM((CPT_DEG, K_DEG), jnp.int32),
        pltpu.VMEM((K_DEG, DHH), _f32),
        pltpu.VMEM_SHARED((N, DHH), _f32),
        pltpu.SemaphoreType.DMA,
        pltpu.SemaphoreType.DMA,
        pltpu.SemaphoreType.DMA,
        pltpu.SemaphoreType.DMA,
    ],
)
def _sc_degree(dst3, ones_tbl, dega, degb, didx, ones_v, acc,
               sm0, sm1, sm2, sm3):
    c = lax.axis_index("c")
    s = lax.axis_index("s")
    sems = (sm0, sm1, sm2, sm3)

    _per_stripe(s, lambda sl: pltpu.sync_copy(ones_tbl.at[sl], acc.at[sl]))
    pltpu.sync_copy(ones_tbl.at[pl.ds(0, K_DEG)], ones_v)
    pltpu.sync_copy(dst3.at[c * NSUB + s], didx)
    plsc.subcore_barrier()

    def fire(u, sem):
        pltpu.async_copy(ones_v, acc.at[didx.at[u]], sem, add=True)

    def drain(u, sem):
        pltpu.make_async_copy(ones_v, acc.at[didx.at[u]], sem).wait()

    def body(g, carry):
        for r in range(4):
            u = 4 * g + r

            @pl.when(u >= 4)
            def _():
                drain(u - 4, sems[r])

            fire(u, sems[r])
        return carry

    lax.fori_loop(0, (CPT_DEG - 1) // 4, body, 0)
    drain(120, sems[0])
    fire(124, sems[0])
    drain(121, sems[1])
    drain(122, sems[2])
    drain(123, sems[3])
    drain(124, sems[0])
    plsc.subcore_barrier()

    @pl.when(c == 0)
    def _():
        _per_stripe(s, lambda sl: pltpu.sync_copy(acc.at[sl], dega.at[sl]))

    @pl.when(c == 1)
    def _():
        _per_stripe(s, lambda sl: pltpu.sync_copy(acc.at[sl], degb.at[sl]))


@functools.partial(
    pl.kernel,
    out_type=(
        jax.ShapeDtypeStruct((N, DHH), _f32),
        jax.ShapeDtypeStruct((N, DHH), _f32),
    ),
    mesh=_mesh,
    scratch_types=[
        pltpu.VMEM((2, 2, KU), jnp.int32),
        pltpu.VMEM((2, 2, KU), jnp.int32),
        pltpu.VMEM((2, KU, DHH), _f32),
        pltpu.VMEM_SHARED((N, DHH), _f32),
        pltpu.SemaphoreType.DMA,
        pltpu.SemaphoreType.DMA,
        pltpu.SemaphoreType.DMA,
        pltpu.SemaphoreType.DMA,
        pltpu.SemaphoreType.DMA,
        pltpu.SemaphoreType.DMA,
    ],
)
def _sc_scatter(src4, dst4, hsa, hsb, outa, outb, sidx, didx, rows, acc,
                is0, is1, gs0, gs1, ss0, ss1):
    c = lax.axis_index("c")
    s = lax.axis_index("s")
    isems = (is0, is1)
    gsems = (gs0, gs1)
    ssems = (ss0, ss1)

    def run(table, out):
        def idx_copy(u, p, r):
            pltpu.async_copy(src4.at[s, u], sidx.at[p, r], isems[r])
            pltpu.async_copy(dst4.at[s, u], didx.at[p, r], isems[r])

        def idx_wait(u, p, r):
            pltpu.make_async_copy(src4.at[s, u], sidx.at[p, r],
                                  isems[r]).wait()
            pltpu.make_async_copy(dst4.at[s, u], didx.at[p, r],
                                  isems[r]).wait()

        def gather(p, r):
            return pltpu.async_copy(table.at[sidx.at[p, r]], rows.at[r],
                                    gsems[r])

        def scat(p, r):
            pltpu.async_copy(rows.at[r], acc.at[didx.at[p, r]], ssems[r],
                             add=True)

        def scat_wait(p, r):
            pltpu.make_async_copy(rows.at[r], acc.at[didx.at[p, r]],
                                  ssems[r]).wait()

        _per_stripe(s, lambda sl: pltpu.sync_copy(table.at[sl], acc.at[sl]))
        idx_copy(0, 0, 0)
        idx_copy(1, 0, 1)
        plsc.subcore_barrier()

        def body(g, carry):
            p = lax.rem(g, 2)
            pn = 1 - p
            u0 = 2 * g
            idx_wait(u0, p, 0)
            d0 = gather(p, 0)

            @pl.when(g >= 1)
            def _():
                scat_wait(pn, 1)

            idx_wait(u0 + 1, p, 1)
            d1 = gather(p, 1)

            @pl.when(u0 + 2 < UNITS)
            def _():
                idx_copy(u0 + 2, pn, 0)
                idx_copy(u0 + 3, pn, 1)

            d0.wait()
            scat(p, 0)
            d1.wait()
            scat(p, 1)
            scat_wait(p, 0)
            return carry

        lax.fori_loop(0, UNITS // 2, body, 0)
        scat_wait((UNITS // 2 - 1) % 2, 1)
        plsc.subcore_barrier()
        _per_stripe(s, lambda sl: pltpu.sync_copy(acc.at[sl], out.at[sl]))

    @pl.when(c == 0)
    def _():
        run(hsa, outa)

    @pl.when(c == 1)
    def _():
        run(hsb, outb)


def _gelu(x):
    return 0.5 * x * (1.0 + lax.erf(x * 0.7071067811865476))


def _graph_norm(t, nw, nb, ms):
    mean = jnp.mean(t, axis=0, keepdims=True)
    xc = t - ms * mean
    var = jnp.mean(xc * xc, axis=0, keepdims=True)
    return nw * xc * lax.rsqrt(var + 1e-5) + nb


def _tc_head_body(x_ref, inw_ref, inb_ref, w0_ref, dega_ref, degb_ref,
                  hsa_ref, hsb_ref, dinv_ref):
    deg = jnp.mean(dega_ref[...] + degb_ref[...], axis=1, keepdims=True) - 1.0
    dinv = lax.rsqrt(jnp.maximum(deg, 1.0))
    dinv = jnp.broadcast_to(dinv, (N, DHH))
    dinv_ref[...] = dinv
    h0 = _gelu(
        jnp.dot(x_ref[...], inw_ref[...], preferred_element_type=_f32)
        + inb_ref[...]
    )
    h2 = jnp.dot(h0, w0_ref[...], preferred_element_type=_f32)
    hsa_ref[...] = h2[:, :DHH] * dinv
    hsb_ref[...] = h2[:, DHH:] * dinv


def _tc_enc_body(acca_ref, accb_ref, dinv_ref, b_ref, nw_ref, nb_ref, ms_ref,
                 w_ref, h_ref, hsa_ref, hsb_ref):
    dinv = dinv_ref[...]
    t = jnp.concatenate([acca_ref[...] * dinv, accb_ref[...] * dinv], axis=1)
    t = t + b_ref[...]
    h = _gelu(_graph_norm(t, nw_ref[...], nb_ref[...], ms_ref[...]))
    h_ref[...] = h
    h2 = jnp.dot(h, w_ref[...], preferred_element_type=_f32)
    hsa_ref[...] = h2[:, :DHH] * dinv
    hsb_ref[...] = h2[:, DHH:] * dinv


def _tc_dec_body(acca_ref, accb_ref, dinv_ref, b_ref, nw_ref, nb_ref, ms_ref,
                 wt_ref, wb_ref, skip_ref, hsa_ref, hsb_ref):
    dinv = dinv_ref[...]
    t = jnp.concatenate([acca_ref[...] * dinv, accb_ref[...] * dinv], axis=1)
    t = t + b_ref[...]
    h = _gelu(_graph_norm(t, nw_ref[...], nb_ref[...], ms_ref[...]))
    h2 = (
        jnp.dot(h, wt_ref[...], preferred_element_type=_f32)
        + jnp.dot(skip_ref[...], wb_ref[...], preferred_element_type=_f32)
    )
    hsa_ref[...] = h2[:, :DHH] * dinv
    hsb_ref[...] = h2[:, DHH:] * dinv


def _tc_tail_body(acca_ref, accb_ref, dinv_ref, b_ref, nw_ref, nb_ref, ms_ref,
                  ow_ref, ob_ref, emb_ref):
    dinv = dinv_ref[...]
    t = jnp.concatenate([acca_ref[...] * dinv, accb_ref[...] * dinv], axis=1)
    t = t + b_ref[...]
    h = _gelu(_graph_norm(t, nw_ref[...], nb_ref[...], ms_ref[...]))
    emb_ref[...] = jnp.tanh(
        jnp.dot(h, ow_ref[...], preferred_element_type=_f32) + ob_ref[...]
    )


def _sds(shape):
    return jax.ShapeDtypeStruct(shape, _f32)


_tc_head = pl.pallas_call(
    _tc_head_body,
    out_shape=(_sds((N, DHH)), _sds((N, DHH)), _sds((N, DHH))),
)
_tc_enc = pl.pallas_call(
    _tc_enc_body,
    out_shape=(_sds((N, DH)), _sds((N, DHH)), _sds((N, DHH))),
)
_tc_dec = pl.pallas_call(
    _tc_dec_body,
    out_shape=(_sds((N, DHH)), _sds((N, DHH))),
)
_tc_tail = pl.pallas_call(
    _tc_tail_body,
    out_shape=_sds((N, DHH)),
)


def kernel(x, edge_index, in_W, in_b, enc_W, enc_b, enc_nw, enc_nb, enc_ms,
           dec_W, dec_b, dec_nw, dec_nb, dec_ms, out_W, out_b):
    src3 = edge_index[0].reshape(NSUB, UNITS, KU)
    dst3 = edge_index[1].reshape(NSUB, UNITS, KU)
    dst3_deg = edge_index[1].reshape(2 * NSUB, CPT_DEG, K_DEG)
    ones_tbl = jnp.ones((N, DHH), _f32)

    dega, degb = _sc_degree(dst3_deg, ones_tbl)

    hsa, hsb, dinv = _tc_head(x, in_W, in_b, enc_W[0], dega, degb)

    skips = []
    for kk in range(1, L + 1):
        acca, accb = _sc_scatter(src3, dst3, hsa, hsb)
        h, hsa, hsb = _tc_enc(
            acca, accb, dinv,
            enc_b[kk - 1], enc_nw[kk - 1], enc_nb[kk - 1], enc_ms[kk - 1],
            enc_W[kk],
        )
        skips.append(h)

    acca, accb = _sc_scatter(src3, dst3, hsa, hsb)
    hsa, hsb = _tc_dec(
        acca, accb, dinv,
        enc_b[L], enc_nw[L], enc_nb[L], enc_ms[L],
        dec_W[0][:DH], dec_W[0][DH:], skips[L - 1],
    )
    for i in range(1, L):
        acca, accb = _sc_scatter(src3, dst3, hsa, hsb)
        hsa, hsb = _tc_dec(
            acca, accb, dinv,
            dec_b[i - 1], dec_nw[i - 1], dec_nb[i - 1], dec_ms[i - 1],
            dec_W[i][:DH], dec_W[i][DH:], skips[L - 1 - i],
        )

    acca, accb = _sc_scatter(src3, dst3, hsa, hsb)
    emb = _tc_tail(
        acca, accb, dinv,
        dec_b[L - 1], dec_nw[L - 1], dec_nb[L - 1], dec_ms[L - 1],
        out_W, out_b,
    )
    return emb


# speedup vs baseline: 1.0467x; 1.0467x over previous
import functools

import jax
import jax.numpy as jnp
from jax import lax
from jax.experimental import pallas as pl
from jax.experimental.pallas import tpu as pltpu
from jax.experimental.pallas import tpu_sc as plsc

N = 10000
E = 320000
DH = 256
DHH = 128
L = 4
NSUB = 16
KU = 100
UNITS = 200
K_DEG = 80
CPT_DEG = 125
STRIPE = 624
STRIPE_LAST = N - 15 * STRIPE

_mesh = plsc.VectorSubcoreMesh(core_axis_name="c", subcore_axis_name="s")
_f32 = jnp.float32


def _per_stripe(s, fn):
    @pl.when(s < 15)
    def _():
        fn(pl.ds(s * STRIPE, STRIPE))

    @pl.when(s == 15)
    def _():
        fn(pl.ds(15 * STRIPE, STRIPE_LAST))


@functools.partial(
    pl.kernel,
    out_type=(
        jax.ShapeDtypeStruct((N, DHH), _f32),
        jax.ShapeDtypeStruct((N, DHH), _f32),
    ),
    mesh=_mesh,
    scratch_types=[
        pltpu.VMEM((CPT_DEG, K_DEG), jnp.int32),
        pltpu.VMEM((K_DEG, DHH), _f32),
        pltpu.VMEM_SHARED((N, DHH), _f32),
        pltpu.SemaphoreType.DMA,
        pltpu.SemaphoreType.DMA,
        pltpu.SemaphoreType.DMA,
        pltpu.SemaphoreType.DMA,
    ],
)
def _sc_degree(dst3, ones_tbl, dega, degb, didx, ones_v, acc,
               sm0, sm1, sm2, sm3):
    c = lax.axis_index("c")
    s = lax.axis_index("s")
    sems = (sm0, sm1, sm2, sm3)

    _per_stripe(s, lambda sl: pltpu.sync_copy(ones_tbl.at[sl], acc.at[sl]))
    pltpu.sync_copy(ones_tbl.at[pl.ds(0, K_DEG)], ones_v)
    pltpu.sync_copy(dst3.at[c * NSUB + s], didx)
    plsc.subcore_barrier()

    def fire(u, sem):
        pltpu.async_copy(ones_v, acc.at[didx.at[u]], sem, add=True)

    def drain(u, sem):
        pltpu.make_async_copy(ones_v, acc.at[didx.at[u]], sem).wait()

    def body(g, carry):
        for r in range(4):
            u = 4 * g + r

            @pl.when(u >= 4)
            def _():
                drain(u - 4, sems[r])

            fire(u, sems[r])
        return carry

    lax.fori_loop(0, (CPT_DEG - 1) // 4, body, 0)
    drain(120, sems[0])
    fire(124, sems[0])
    drain(121, sems[1])
    drain(122, sems[2])
    drain(123, sems[3])
    drain(124, sems[0])
    plsc.subcore_barrier()

    @pl.when(c == 0)
    def _():
        _per_stripe(s, lambda sl: pltpu.sync_copy(acc.at[sl], dega.at[sl]))

    @pl.when(c == 1)
    def _():
        _per_stripe(s, lambda sl: pltpu.sync_copy(acc.at[sl], degb.at[sl]))


@functools.partial(
    pl.kernel,
    out_type=(
        jax.ShapeDtypeStruct((N, DHH), _f32),
        jax.ShapeDtypeStruct((N, DHH), _f32),
    ),
    mesh=_mesh,
    scratch_types=[
        pltpu.VMEM((4, KU), jnp.int32),
        pltpu.VMEM((4, KU), jnp.int32),
        pltpu.VMEM((2, KU, DHH), _f32),
        pltpu.VMEM_SHARED((N, DHH), _f32),
        pltpu.SemaphoreType.DMA,
        pltpu.SemaphoreType.DMA,
        pltpu.SemaphoreType.DMA,
        pltpu.SemaphoreType.DMA,
        pltpu.SemaphoreType.DMA,
        pltpu.SemaphoreType.DMA,
        pltpu.SemaphoreType.DMA,
        pltpu.SemaphoreType.DMA,
    ],
)
def _sc_scatter(src4, dst4, hsa, hsb, outa, outb, sidx, didx, rows, acc,
                is0, is1, is2, is3, gs0, gs1, ss0, ss1):
    c = lax.axis_index("c")
    s = lax.axis_index("s")
    isems = (is0, is1, is2, is3)
    gsems = (gs0, gs1)
    ssems = (ss0, ss1)

    def run(table, out):
        def idx_copy(u, q):
            pltpu.async_copy(src4.at[s, u], sidx.at[q], isems[q])
            pltpu.async_copy(dst4.at[s, u], didx.at[q], isems[q])

        def idx_wait(u, q):
            pltpu.make_async_copy(src4.at[s, u], sidx.at[q], isems[q]).wait()
            pltpu.make_async_copy(dst4.at[s, u], didx.at[q], isems[q]).wait()

        def gather_now(q, b):
            pltpu.async_copy(table.at[sidx.at[q]], rows.at[b],
                             gsems[b]).wait()

        def scat(q, b):
            pltpu.async_copy(rows.at[b], acc.at[didx.at[q]], ssems[b],
                             add=True)

        def scat_wait(q, b):
            pltpu.make_async_copy(rows.at[b], acc.at[didx.at[q]],
                                  ssems[b]).wait()

        _per_stripe(s, lambda sl: pltpu.sync_copy(table.at[sl], acc.at[sl]))
        idx_copy(0, 0)
        idx_copy(1, 1)
        plsc.subcore_barrier()

        def step(u, q, b):
            @pl.when(u >= 2)
            def _():
                scat_wait((q + 2) % 4, b)

            idx_wait(u, q)
            gather_now(q, b)
            scat(q, b)

            @pl.when(u + 2 < UNITS)
            def _():
                idx_copy(u + 2, (q + 2) % 4)

        def body(g, carry):
            for r in range(4):
                step(4 * g + r, r, r & 1)
            return carry

        _tail_start = ((UNITS - 2) // 4) * 4
        lax.fori_loop(0, _tail_start // 4, body, 0)
        for _u in range(_tail_start, UNITS):
            step(_u, _u % 4, _u % 2)
        scat_wait((UNITS - 2) % 4, (UNITS - 2) % 2)
        scat_wait((UNITS - 1) % 4, (UNITS - 1) % 2)
        plsc.subcore_barrier()
        _per_stripe(s, lambda sl: pltpu.sync_copy(acc.at[sl], out.at[sl]))

    @pl.when(c == 0)
    def _():
        run(hsa, outa)

    @pl.when(c == 1)
    def _():
        run(hsb, outb)


def _gelu(x):
    return 0.5 * x * (1.0 + lax.erf(x * 0.7071067811865476))


def _graph_norm(t, nw, nb, ms):
    mean = jnp.mean(t, axis=0, keepdims=True)
    xc = t - ms * mean
    var = jnp.mean(xc * xc, axis=0, keepdims=True)
    return nw * xc * lax.rsqrt(var + 1e-5) + nb


def _tc_head_body(x_ref, inw_ref, inb_ref, w0_ref, dega_ref, degb_ref,
                  hsa_ref, hsb_ref, dinv_ref):
    deg = jnp.mean(dega_ref[...] + degb_ref[...], axis=1, keepdims=True) - 1.0
    dinv = lax.rsqrt(jnp.maximum(deg, 1.0))
    dinv = jnp.broadcast_to(dinv, (N, DHH))
    dinv_ref[...] = dinv
    h0 = _gelu(
        jnp.dot(x_ref[...], inw_ref[...], preferred_element_type=_f32)
        + inb_ref[...]
    )
    h2 = jnp.dot(h0, w0_ref[...], preferred_element_type=_f32)
    hsa_ref[...] = h2[:, :DHH] * dinv
    hsb_ref[...] = h2[:, DHH:] * dinv


def _tc_enc_body(acca_ref, accb_ref, dinv_ref, b_ref, nw_ref, nb_ref, ms_ref,
                 w_ref, h_ref, hsa_ref, hsb_ref):
    dinv = dinv_ref[...]
    t = jnp.concatenate([acca_ref[...] * dinv, accb_ref[...] * dinv], axis=1)
    t = t + b_ref[...]
    h = _gelu(_graph_norm(t, nw_ref[...], nb_ref[...], ms_ref[...]))
    h_ref[...] = h
    h2 = jnp.dot(h, w_ref[...], preferred_element_type=_f32)
    hsa_ref[...] = h2[:, :DHH] * dinv
    hsb_ref[...] = h2[:, DHH:] * dinv


def _tc_dec_body(acca_ref, accb_ref, dinv_ref, b_ref, nw_ref, nb_ref, ms_ref,
                 wt_ref, wb_ref, skip_ref, hsa_ref, hsb_ref):
    dinv = dinv_ref[...]
    t = jnp.concatenate([acca_ref[...] * dinv, accb_ref[...] * dinv], axis=1)
    t = t + b_ref[...]
    h = _gelu(_graph_norm(t, nw_ref[...], nb_ref[...], ms_ref[...]))
    h2 = (
        jnp.dot(h, wt_ref[...], preferred_element_type=_f32)
        + jnp.dot(skip_ref[...], wb_ref[...], preferred_element_type=_f32)
    )
    hsa_ref[...] = h2[:, :DHH] * dinv
    hsb_ref[...] = h2[:, DHH:] * dinv


def _tc_tail_body(acca_ref, accb_ref, dinv_ref, b_ref, nw_ref, nb_ref, ms_ref,
                  ow_ref, ob_ref, emb_ref):
    dinv = dinv_ref[...]
    t = jnp.concatenate([acca_ref[...] * dinv, accb_ref[...] * dinv], axis=1)
    t = t + b_ref[...]
    h = _gelu(_graph_norm(t, nw_ref[...], nb_ref[...], ms_ref[...]))
    emb_ref[...] = jnp.tanh(
        jnp.dot(h, ow_ref[...], preferred_element_type=_f32) + ob_ref[...]
    )


def _sds(shape):
    return jax.ShapeDtypeStruct(shape, _f32)


_tc_head = pl.pallas_call(
    _tc_head_body,
    out_shape=(_sds((N, DHH)), _sds((N, DHH)), _sds((N, DHH))),
)
_tc_enc = pl.pallas_call(
    _tc_enc_body,
    out_shape=(_sds((N, DH)), _sds((N, DHH)), _sds((N, DHH))),
)
_tc_dec = pl.pallas_call(
    _tc_dec_body,
    out_shape=(_sds((N, DHH)), _sds((N, DHH))),
)
_tc_tail = pl.pallas_call(
    _tc_tail_body,
    out_shape=_sds((N, DHH)),
)


def kernel(x, edge_index, in_W, in_b, enc_W, enc_b, enc_nw, enc_nb, enc_ms,
           dec_W, dec_b, dec_nw, dec_nb, dec_ms, out_W, out_b):
    src3 = edge_index[0].reshape(NSUB, UNITS, KU)
    dst3 = edge_index[1].reshape(NSUB, UNITS, KU)
    dst3_deg = edge_index[1].reshape(2 * NSUB, CPT_DEG, K_DEG)
    ones_tbl = jnp.ones((N, DHH), _f32)

    dega, degb = _sc_degree(dst3_deg, ones_tbl)

    hsa, hsb, dinv = _tc_head(x, in_W, in_b, enc_W[0], dega, degb)

    skips = []
    for kk in range(1, L + 1):
        acca, accb = _sc_scatter(src3, dst3, hsa, hsb)
        h, hsa, hsb = _tc_enc(
            acca, accb, dinv,
            enc_b[kk - 1], enc_nw[kk - 1], enc_nb[kk - 1], enc_ms[kk - 1],
            enc_W[kk],
        )
        skips.append(h)

    acca, accb = _sc_scatter(src3, dst3, hsa, hsb)
    hsa, hsb = _tc_dec(
        acca, accb, dinv,
        enc_b[L], enc_nw[L], enc_nb[L], enc_ms[L],
        dec_W[0][:DH], dec_W[0][DH:], skips[L - 1],
    )
    for i in range(1, L):
        acca, accb = _sc_scatter(src3, dst3, hsa, hsb)
        hsa, hsb = _tc_dec(
            acca, accb, dinv,
            dec_b[i - 1], dec_nw[i - 1], dec_nb[i - 1], dec_ms[i - 1],
            dec_W[i][:DH], dec_W[i][DH:], skips[L - 1 - i],
        )

    acca, accb = _sc_scatter(src3, dst3, hsa, hsb)
    emb = _tc_tail(
        acca, accb, dinv,
        dec_b[L - 1], dec_nw[L - 1], dec_nb[L - 1], dec_ms[L - 1],
        out_W, out_b,
    )
    return emb
